# baseline (speedup 1.0000x reference)
import functools

import jax
import jax.numpy as jnp
from jax import lax
from jax.experimental import pallas as pl
from jax.experimental.pallas import tpu as pltpu
from jax.experimental.pallas import tpu_sc as plsc

B = 16384
LSEQ = 50
V = 1000000
D = 64
NC = 2
NS = 16
NW = NC * NS
RPT = B // NW
CHUNKS = RPT // 2
INV_L = 0.02

_BV = 4096


def _fuse_transpose_body(a_ref, b_ref, o_ref):
  o_ref[...] = jnp.concatenate([a_ref[...].T, b_ref[...].T], axis=1)


def _fuse_transpose(ta_t, tb_t):
  grid = (V + _BV - 1) // _BV
  in_spec = pl.BlockSpec((D, _BV), lambda g: (0, g))
  return pl.pallas_call(
      _fuse_transpose_body,
      grid=(grid,),
      in_specs=[in_spec, in_spec],
      out_specs=pl.BlockSpec((_BV, 2 * D), lambda g: (g, 0)),
      out_shape=jax.ShapeDtypeStruct((V, 2 * D), jnp.float32),
  )(ta_t, tb_t)


def _emb_bag(idx_pad, fused):
  mesh = plsc.VectorSubcoreMesh(
      core_axis_name="c", subcore_axis_name="s", num_cores=NC,
      num_subcores=NS)

  @functools.partial(
      pl.kernel,
      out_type=jax.ShapeDtypeStruct((B, 2 * D), jnp.float32),
      mesh=mesh,
      scratch_types=[
          pltpu.VMEM((CHUNKS, 128), jnp.int32),
          pltpu.VMEM((2, 100, 2 * D), jnp.float32),
          pltpu.VMEM((RPT, 2 * D), jnp.float32),
          pltpu.SemaphoreType.DMA,
          pltpu.SemaphoreType.DMA,
      ],
  )
  def bag(idx_hbm, tbl, out, idx_v, rows, stage, sem0, sem1):
    wid = lax.axis_index("s") * NC + lax.axis_index("c")
    ibase = wid * CHUNKS
    obase = wid * RPT
    sems = (sem0, sem1)

    def reduce_bag(par, c):
      for half in range(2):
        r0 = LSEQ * half
        acc = tuple(rows[par, r0, pl.ds(16 * k, 16)] for k in range(8))

        def red7(j, accs, _r0=r0, _par=par):
          r = _r0 + 1 + j * 7
          for t in range(7):
            accs = tuple(
                accs[k] + rows[_par, r + t, pl.ds(16 * k, 16)]
                for k in range(8))
          return accs

        acc = lax.fori_loop(0, 7, red7, acc)
        row_out = 2 * c + half
        for k in range(8):
          stage[row_out, pl.ds(16 * k, 16)] = acc[k] * INV_L

    pltpu.sync_copy(idx_hbm.at[pl.ds(ibase, CHUNKS)], idx_v)

    def start(c, par):
      pltpu.async_copy(
          tbl.at[idx_v.at[c, pl.ds(0, 100)]], rows.at[par], sems[par])

    start(0, 0)
    start(1, 1)

    def body(i, carry):
      for par in range(2):
        c = 2 * i + par
        pltpu.make_async_copy(
            tbl.at[idx_v.at[c, pl.ds(0, 100)]], rows.at[par],
            sems[par]).wait()
        reduce_bag(par, c)

        @pl.when(c + 2 < CHUNKS)
        def _():
          start(c + 2, par)

      return carry

    lax.fori_loop(0, CHUNKS // 2, body, 0)
    pltpu.sync_copy(stage, out.at[pl.ds(obase, RPT)])

  return bag(idx_pad, fused)


_BLK = 2048


def _towers_body(up_ref, ip_ref, w1t_ref, b1_ref, w2t_ref, b2_ref, wat_ref,
                 ba_ref, out_ref):
  up = up_ref[...]
  ip = ip_ref[...]
  mlp = jnp.concatenate([up[:, :D], ip[:, :D]], axis=1)
  h1 = jnp.maximum(
      jnp.dot(mlp, w1t_ref[...], preferred_element_type=jnp.float32)
      + b1_ref[...], 0.0)
  h2 = jnp.maximum(
      jnp.dot(h1, w2t_ref[...], preferred_element_type=jnp.float32)
      + b2_ref[...], 0.0)
  gmf = up[:, D:] * ip[:, D:]
  vec = jnp.concatenate([h2, gmf], axis=1)
  out_ref[...] = (
      jnp.dot(vec, wat_ref[...], preferred_element_type=jnp.float32)
      + ba_ref[...])


def _towers(up, ip, W1, b1, W2, b2, Wa, ba):
  grid = B // _BLK
  emb_spec = pl.BlockSpec((_BLK, 2 * D), lambda g: (g, 0))
  full = lambda shape: pl.BlockSpec(shape, lambda g: (0, 0))
  return pl.pallas_call(
      _towers_body,
      grid=(grid,),
      in_specs=[
          emb_spec, emb_spec,
          full((128, 64)), full((1, 64)),
          full((64, 32)), full((1, 32)),
          full((96, 1)), full((1, 1)),
      ],
      out_specs=pl.BlockSpec((_BLK, 1), lambda g: (g, 0)),
      out_shape=jax.ShapeDtypeStruct((B, 1), jnp.float32),
  )(up, ip, W1.T, b1.reshape(1, 64), W2.T, b2.reshape(1, 32),
    Wa.T, ba.reshape(1, 1))


def kernel(usr_comments, descriptions, emb_user_mlp, emb_item_mlp,
           emb_user_gmf, emb_item_gmf, W1, b1, W2, b2, Wa, ba):
  usr_pad = jnp.pad(usr_comments.reshape(B // 2, 2 * LSEQ),
                    ((0, 0), (0, 28)))
  desc_pad = jnp.pad(descriptions.reshape(B // 2, 2 * LSEQ),
                     ((0, 0), (0, 28)))
  fused_u = _fuse_transpose(emb_user_mlp.T, emb_user_gmf.T)
  fused_i = _fuse_transpose(emb_item_mlp.T, emb_item_gmf.T)
  up = _emb_bag(usr_pad, fused_u)
  ip = _emb_bag(desc_pad, fused_i)
  return _towers(up, ip, W1, b1, W2, b2, Wa, ba)


# speedup vs baseline: 1.0961x; 1.0961x over previous
import functools

import jax
import jax.numpy as jnp
from jax import lax
from jax.experimental import pallas as pl
from jax.experimental.pallas import tpu as pltpu
from jax.experimental.pallas import tpu_sc as plsc

B = 16384
LSEQ = 50
V = 1000000
D = 64
NC = 2
NS = 16
NW = NC * NS
RPT = B // NW
CHUNKS = RPT // 2
INV_L = 0.02

_BV = 8192


def _fuse_transpose_body(a_ref, b_ref, o_ref):
  o_ref[:, :D] = a_ref[...].T
  o_ref[:, D:] = b_ref[...].T


def _fuse_transpose(ta_t, tb_t):
  grid = (V + _BV - 1) // _BV
  in_spec = pl.BlockSpec((D, _BV), lambda g: (0, g))
  return pl.pallas_call(
      _fuse_transpose_body,
      grid=(grid,),
      in_specs=[in_spec, in_spec],
      out_specs=pl.BlockSpec((_BV, 2 * D), lambda g: (g, 0)),
      out_shape=jax.ShapeDtypeStruct((V, 2 * D), jnp.float32),
  )(ta_t, tb_t)


def _emb_bag(idx_pad, fused):
  mesh = plsc.VectorSubcoreMesh(
      core_axis_name="c", subcore_axis_name="s", num_cores=NC,
      num_subcores=NS)

  @functools.partial(
      pl.kernel,
      out_type=jax.ShapeDtypeStruct((B, 2 * D), jnp.float32),
      mesh=mesh,
      scratch_types=[
          pltpu.VMEM((CHUNKS, 128), jnp.int32),
          pltpu.VMEM((2, 100, 2 * D), jnp.float32),
          pltpu.VMEM((RPT, 2 * D), jnp.float32),
          pltpu.SemaphoreType.DMA,
          pltpu.SemaphoreType.DMA,
      ],
  )
  def bag(idx_hbm, tbl, out, idx_v, rows, stage, sem0, sem1):
    wid = lax.axis_index("s") * NC + lax.axis_index("c")
    ibase = wid * CHUNKS
    obase = wid * RPT
    sems = (sem0, sem1)

    def reduce_bag(par, c):
      for half in range(2):
        r0 = LSEQ * half
        acc = tuple(rows[par, r0, pl.ds(16 * k, 16)] for k in range(8))

        def red7(j, accs, _r0=r0, _par=par):
          r = _r0 + 1 + j * 7
          for t in range(7):
            accs = tuple(
                accs[k] + rows[_par, r + t, pl.ds(16 * k, 16)]
                for k in range(8))
          return accs

        acc = lax.fori_loop(0, 7, red7, acc)
        row_out = 2 * c + half
        for k in range(8):
          stage[row_out, pl.ds(16 * k, 16)] = acc[k] * INV_L

    pltpu.sync_copy(idx_hbm.at[pl.ds(ibase, CHUNKS)], idx_v)

    def start(c, par):
      pltpu.async_copy(
          tbl.at[idx_v.at[c, pl.ds(0, 100)]], rows.at[par], sems[par])

    start(0, 0)
    start(1, 1)

    def body(i, carry):
      for par in range(2):
        c = 2 * i + par
        pltpu.make_async_copy(
            tbl.at[idx_v.at[c, pl.ds(0, 100)]], rows.at[par],
            sems[par]).wait()
        reduce_bag(par, c)

        @pl.when(c + 2 < CHUNKS)
        def _():
          start(c + 2, par)

      return carry

    lax.fori_loop(0, CHUNKS // 2, body, 0)
    pltpu.sync_copy(stage, out.at[pl.ds(obase, RPT)])

  return bag(idx_pad, fused)


_BLK = 2048


def _towers_body(up_ref, ip_ref, w1t_ref, b1_ref, w2t_ref, b2_ref, wat_ref,
                 ba_ref, out_ref):
  up = up_ref[...]
  ip = ip_ref[...]
  mlp = jnp.concatenate([up[:, :D], ip[:, :D]], axis=1)
  h1 = jnp.maximum(
      jnp.dot(mlp, w1t_ref[...], preferred_element_type=jnp.float32)
      + b1_ref[...], 0.0)
  h2 = jnp.maximum(
      jnp.dot(h1, w2t_ref[...], preferred_element_type=jnp.float32)
      + b2_ref[...], 0.0)
  gmf = up[:, D:] * ip[:, D:]
  vec = jnp.concatenate([h2, gmf], axis=1)
  out_ref[...] = (
      jnp.dot(vec, wat_ref[...], preferred_element_type=jnp.float32)
      + ba_ref[...])


def _towers(up, ip, W1, b1, W2, b2, Wa, ba):
  grid = B // _BLK
  emb_spec = pl.BlockSpec((_BLK, 2 * D), lambda g: (g, 0))
  full = lambda shape: pl.BlockSpec(shape, lambda g: (0, 0))
  return pl.pallas_call(
      _towers_body,
      grid=(grid,),
      in_specs=[
          emb_spec, emb_spec,
          full((128, 64)), full((1, 64)),
          full((64, 32)), full((1, 32)),
          full((96, 1)), full((1, 1)),
      ],
      out_specs=pl.BlockSpec((_BLK, 1), lambda g: (g, 0)),
      out_shape=jax.ShapeDtypeStruct((B, 1), jnp.float32),
  )(up, ip, W1.T, b1.reshape(1, 64), W2.T, b2.reshape(1, 32),
    Wa.T, ba.reshape(1, 1))


def kernel(usr_comments, descriptions, emb_user_mlp, emb_item_mlp,
           emb_user_gmf, emb_item_gmf, W1, b1, W2, b2, Wa, ba):
  usr_pad = jnp.pad(usr_comments.reshape(B // 2, 2 * LSEQ),
                    ((0, 0), (0, 28)))
  desc_pad = jnp.pad(descriptions.reshape(B // 2, 2 * LSEQ),
                     ((0, 0), (0, 28)))
  fused_u = _fuse_transpose(emb_user_mlp.T, emb_user_gmf.T)
  fused_i = _fuse_transpose(emb_item_mlp.T, emb_item_gmf.T)
  up = _emb_bag(usr_pad, fused_u)
  ip = _emb_bag(desc_pad, fused_i)
  return _towers(up, ip, W1, b1, W2, b2, Wa, ba)


# speedup vs baseline: 1.1325x; 1.0332x over previous
import functools

import jax
import jax.numpy as jnp
from jax import lax
from jax.experimental import pallas as pl
from jax.experimental.pallas import tpu as pltpu
from jax.experimental.pallas import tpu_sc as plsc

B = 16384
LSEQ = 50
V = 1000000
D = 64
NC = 2
NS = 16
NW = NC * NS
RPT = B // NW
CHUNKS = RPT // 2
INV_L = 0.02

_BV = 16384


def _fuse_transpose_body(a_ref, b_ref, o_ref):
  o_ref[:, :D] = a_ref[...].T
  o_ref[:, D:] = b_ref[...].T


def _fuse_transpose(ta_t, tb_t):
  grid = (V + _BV - 1) // _BV
  in_spec = pl.BlockSpec((D, _BV), lambda g: (0, g))
  return pl.pallas_call(
      _fuse_transpose_body,
      grid=(grid,),
      in_specs=[in_spec, in_spec],
      out_specs=pl.BlockSpec((_BV, 2 * D), lambda g: (g, 0)),
      out_shape=jax.ShapeDtypeStruct((V, 2 * D), jnp.float32),
  )(ta_t, tb_t)


def _emb_bag(idx_pad, fused):
  mesh = plsc.VectorSubcoreMesh(
      core_axis_name="c", subcore_axis_name="s", num_cores=NC,
      num_subcores=NS)

  @functools.partial(
      pl.kernel,
      out_type=jax.ShapeDtypeStruct((B, 2 * D), jnp.float32),
      mesh=mesh,
      scratch_types=[
          pltpu.VMEM((CHUNKS, 128), jnp.int32),
          pltpu.VMEM((2, 100, 2 * D), jnp.float32),
          pltpu.VMEM((RPT, 2 * D), jnp.float32),
          pltpu.SemaphoreType.DMA,
          pltpu.SemaphoreType.DMA,
      ],
  )
  def bag(idx_hbm, tbl, out, idx_v, rows, stage, sem0, sem1):
    wid = lax.axis_index("s") * NC + lax.axis_index("c")
    ibase = wid * CHUNKS
    obase = wid * RPT
    sems = (sem0, sem1)

    def reduce_bag(par, c):
      for half in range(2):
        r0 = LSEQ * half
        acc = tuple(rows[par, r0, pl.ds(16 * k, 16)] for k in range(8))

        def red7(j, accs, _r0=r0, _par=par):
          r = _r0 + 1 + j * 7
          for t in range(7):
            accs = tuple(
                accs[k] + rows[_par, r + t, pl.ds(16 * k, 16)]
                for k in range(8))
          return accs

        acc = lax.fori_loop(0, 7, red7, acc)
        row_out = 2 * c + half
        for k in range(8):
          stage[row_out, pl.ds(16 * k, 16)] = acc[k] * INV_L

    pltpu.sync_copy(idx_hbm.at[pl.ds(ibase, CHUNKS)], idx_v)

    def start(c, par):
      pltpu.async_copy(
          tbl.at[idx_v.at[c, pl.ds(0, 100)]], rows.at[par], sems[par])

    start(0, 0)
    start(1, 1)

    def body(i, carry):
      for par in range(2):
        c = 2 * i + par
        pltpu.make_async_copy(
            tbl.at[idx_v.at[c, pl.ds(0, 100)]], rows.at[par],
            sems[par]).wait()
        reduce_bag(par, c)

        @pl.when(c + 2 < CHUNKS)
        def _():
          start(c + 2, par)

      return carry

    lax.fori_loop(0, CHUNKS // 2, body, 0)
    pltpu.sync_copy(stage, out.at[pl.ds(obase, RPT)])

  return bag(idx_pad, fused)


_BLK = 2048


def _towers_body(up_ref, ip_ref, w1t_ref, b1_ref, w2t_ref, b2_ref, wat_ref,
                 ba_ref, out_ref):
  up = up_ref[...]
  ip = ip_ref[...]
  mlp = jnp.concatenate([up[:, :D], ip[:, :D]], axis=1)
  h1 = jnp.maximum(
      jnp.dot(mlp, w1t_ref[...], preferred_element_type=jnp.float32)
      + b1_ref[...], 0.0)
  h2 = jnp.maximum(
      jnp.dot(h1, w2t_ref[...], preferred_element_type=jnp.float32)
      + b2_ref[...], 0.0)
  gmf = up[:, D:] * ip[:, D:]
  vec = jnp.concatenate([h2, gmf], axis=1)
  out_ref[...] = (
      jnp.dot(vec, wat_ref[...], preferred_element_type=jnp.float32)
      + ba_ref[...])


def _towers(up, ip, W1, b1, W2, b2, Wa, ba):
  grid = B // _BLK
  emb_spec = pl.BlockSpec((_BLK, 2 * D), lambda g: (g, 0))
  full = lambda shape: pl.BlockSpec(shape, lambda g: (0, 0))
  return pl.pallas_call(
      _towers_body,
      grid=(grid,),
      in_specs=[
          emb_spec, emb_spec,
          full((128, 64)), full((1, 64)),
          full((64, 32)), full((1, 32)),
          full((96, 1)), full((1, 1)),
      ],
      out_specs=pl.BlockSpec((_BLK, 1), lambda g: (g, 0)),
      out_shape=jax.ShapeDtypeStruct((B, 1), jnp.float32),
  )(up, ip, W1.T, b1.reshape(1, 64), W2.T, b2.reshape(1, 32),
    Wa.T, ba.reshape(1, 1))


def kernel(usr_comments, descriptions, emb_user_mlp, emb_item_mlp,
           emb_user_gmf, emb_item_gmf, W1, b1, W2, b2, Wa, ba):
  usr_pad = jnp.pad(usr_comments.reshape(B // 2, 2 * LSEQ),
                    ((0, 0), (0, 28)))
  desc_pad = jnp.pad(descriptions.reshape(B // 2, 2 * LSEQ),
                     ((0, 0), (0, 28)))
  fused_u = _fuse_transpose(emb_user_mlp.T, emb_user_gmf.T)
  fused_i = _fuse_transpose(emb_item_mlp.T, emb_item_gmf.T)
  up = _emb_bag(usr_pad, fused_u)
  ip = _emb_bag(desc_pad, fused_i)
  return _towers(up, ip, W1, b1, W2, b2, Wa, ba)


# speedup vs baseline: 1.1327x; 1.0002x over previous
import functools

import jax
import jax.numpy as jnp
from jax import lax
from jax.experimental import pallas as pl
from jax.experimental.pallas import tpu as pltpu
from jax.experimental.pallas import tpu_sc as plsc

B = 16384
LSEQ = 50
V = 1000000
D = 64
NC = 2
NS = 16
NW = NC * NS
RPT = B // NW
CHUNKS = RPT // 2
INV_L = 0.02

_BV = 16384
_BVSUB = 2048


def _fuse_transpose_body(a_ref, b_ref, o_ref):
  for s in range(_BV // _BVSUB):
    sl = pl.ds(s * _BVSUB, _BVSUB)
    o_ref[sl, :D] = a_ref[:, sl].T
    o_ref[sl, D:] = b_ref[:, sl].T


def _fuse_transpose(ta_t, tb_t):
  grid = (V + _BV - 1) // _BV
  in_spec = pl.BlockSpec((D, _BV), lambda g: (0, g))
  return pl.pallas_call(
      _fuse_transpose_body,
      grid=(grid,),
      in_specs=[in_spec, in_spec],
      out_specs=pl.BlockSpec((_BV, 2 * D), lambda g: (g, 0)),
      out_shape=jax.ShapeDtypeStruct((V, 2 * D), jnp.float32),
  )(ta_t, tb_t)


def _emb_bag(idx_pad, fused):
  mesh = plsc.VectorSubcoreMesh(
      core_axis_name="c", subcore_axis_name="s", num_cores=NC,
      num_subcores=NS)

  @functools.partial(
      pl.kernel,
      out_type=jax.ShapeDtypeStruct((B, 2 * D), jnp.float32),
      mesh=mesh,
      scratch_types=[
          pltpu.VMEM((CHUNKS, 128), jnp.int32),
          pltpu.VMEM((2, 100, 2 * D), jnp.float32),
          pltpu.VMEM((RPT, 2 * D), jnp.float32),
          pltpu.SemaphoreType.DMA,
          pltpu.SemaphoreType.DMA,
      ],
  )
  def bag(idx_hbm, tbl, out, idx_v, rows, stage, sem0, sem1):
    wid = lax.axis_index("s") * NC + lax.axis_index("c")
    ibase = wid * CHUNKS
    obase = wid * RPT
    sems = (sem0, sem1)

    def reduce_bag(par, c):
      for half in range(2):
        r0 = LSEQ * half
        acc = tuple(rows[par, r0, pl.ds(16 * k, 16)] for k in range(8))

        def red7(j, accs, _r0=r0, _par=par):
          r = _r0 + 1 + j * 7
          for t in range(7):
            accs = tuple(
                accs[k] + rows[_par, r + t, pl.ds(16 * k, 16)]
                for k in range(8))
          return accs

        acc = lax.fori_loop(0, 7, red7, acc)
        row_out = 2 * c + half
        for k in range(8):
          stage[row_out, pl.ds(16 * k, 16)] = acc[k] * INV_L

    pltpu.sync_copy(idx_hbm.at[pl.ds(ibase, CHUNKS)], idx_v)

    def start(c, par):
      pltpu.async_copy(
          tbl.at[idx_v.at[c, pl.ds(0, 100)]], rows.at[par], sems[par])

    start(0, 0)
    start(1, 1)

    def body(i, carry):
      for par in range(2):
        c = 2 * i + par
        pltpu.make_async_copy(
            tbl.at[idx_v.at[c, pl.ds(0, 100)]], rows.at[par],
            sems[par]).wait()
        reduce_bag(par, c)

        @pl.when(c + 2 < CHUNKS)
        def _():
          start(c + 2, par)

      return carry

    lax.fori_loop(0, CHUNKS // 2, body, 0)
    pltpu.sync_copy(stage, out.at[pl.ds(obase, RPT)])

  return bag(idx_pad, fused)


_BLK = 2048


def _towers_body(up_ref, ip_ref, w1t_ref, b1_ref, w2t_ref, b2_ref, wat_ref,
                 ba_ref, out_ref):
  up = up_ref[...]
  ip = ip_ref[...]
  mlp = jnp.concatenate([up[:, :D], ip[:, :D]], axis=1)
  h1 = jnp.maximum(
      jnp.dot(mlp, w1t_ref[...], preferred_element_type=jnp.float32)
      + b1_ref[...], 0.0)
  h2 = jnp.maximum(
      jnp.dot(h1, w2t_ref[...], preferred_element_type=jnp.float32)
      + b2_ref[...], 0.0)
  gmf = up[:, D:] * ip[:, D:]
  vec = jnp.concatenate([h2, gmf], axis=1)
  out_ref[...] = (
      jnp.dot(vec, wat_ref[...], preferred_element_type=jnp.float32)
      + ba_ref[...])


def _towers(up, ip, W1, b1, W2, b2, Wa, ba):
  grid = B // _BLK
  emb_spec = pl.BlockSpec((_BLK, 2 * D), lambda g: (g, 0))
  full = lambda shape: pl.BlockSpec(shape, lambda g: (0, 0))
  return pl.pallas_call(
      _towers_body,
      grid=(grid,),
      in_specs=[
          emb_spec, emb_spec,
          full((128, 64)), full((1, 64)),
          full((64, 32)), full((1, 32)),
          full((96, 1)), full((1, 1)),
      ],
      out_specs=pl.BlockSpec((_BLK, 1), lambda g: (g, 0)),
      out_shape=jax.ShapeDtypeStruct((B, 1), jnp.float32),
  )(up, ip, W1.T, b1.reshape(1, 64), W2.T, b2.reshape(1, 32),
    Wa.T, ba.reshape(1, 1))


def kernel(usr_comments, descriptions, emb_user_mlp, emb_item_mlp,
           emb_user_gmf, emb_item_gmf, W1, b1, W2, b2, Wa, ba):
  usr_pad = jnp.pad(usr_comments.reshape(B // 2, 2 * LSEQ),
                    ((0, 0), (0, 28)))
  desc_pad = jnp.pad(descriptions.reshape(B // 2, 2 * LSEQ),
                     ((0, 0), (0, 28)))
  fused_u = _fuse_transpose(emb_user_mlp.T, emb_user_gmf.T)
  fused_i = _fuse_transpose(emb_item_mlp.T, emb_item_gmf.T)
  up = _emb_bag(usr_pad, fused_u)
  ip = _emb_bag(desc_pad, fused_i)
  return _towers(up, ip, W1, b1, W2, b2, Wa, ba)


# speedup vs baseline: 1.2067x; 1.0653x over previous
import functools

import jax
import jax.numpy as jnp
from jax import lax
from jax.experimental import pallas as pl
from jax.experimental.pallas import tpu as pltpu
from jax.experimental.pallas import tpu_sc as plsc

B = 16384
LSEQ = 50
V = 1000000
D = 64
NC = 2
NS = 16
NW = NC * NS
RPT = B // NW
CHUNKS = RPT // 2
INV_L = 0.02

_BV = 16384
_BVSUB = 2048


def _fuse_transpose_body(a_ref, b_ref, o_ref):
  for s in range(_BV // _BVSUB):
    sl = pl.ds(s * _BVSUB, _BVSUB)
    o_ref[sl, :D] = a_ref[:, sl].T
    o_ref[sl, D:] = b_ref[:, sl].T


def _fuse_transpose(ta_t, tb_t):
  grid = (V + _BV - 1) // _BV
  in_spec = pl.BlockSpec((D, _BV), lambda g: (0, g))
  return pl.pallas_call(
      _fuse_transpose_body,
      grid=(grid,),
      in_specs=[in_spec, in_spec],
      out_specs=pl.BlockSpec((_BV, 2 * D), lambda g: (g, 0)),
      out_shape=jax.ShapeDtypeStruct((V, 2 * D), jnp.float32),
  )(ta_t, tb_t)


def _emb_bag(idx_pad, fused):
  mesh = plsc.VectorSubcoreMesh(
      core_axis_name="c", subcore_axis_name="s", num_cores=NC,
      num_subcores=NS)

  @functools.partial(
      pl.kernel,
      out_type=jax.ShapeDtypeStruct((B, 2 * D), jnp.float32),
      mesh=mesh,
      scratch_types=[
          pltpu.VMEM((CHUNKS, 128), jnp.int32),
          pltpu.VMEM((4, 100, 2 * D), jnp.float32),
          pltpu.VMEM((128, 2 * D), jnp.float32),
          pltpu.SemaphoreType.DMA,
          pltpu.SemaphoreType.DMA,
          pltpu.SemaphoreType.DMA,
          pltpu.SemaphoreType.DMA,
      ],
  )
  def bag(idx_hbm, tbl, out, idx_v, rows, stage, sem0, sem1, sem2, sem3):
    wid = lax.axis_index("s") * NC + lax.axis_index("c")
    ibase = wid * CHUNKS
    obase = wid * RPT
    sems = (sem0, sem1, sem2, sem3)

    def reduce_bag(par, c_l):
      for half in range(2):
        r0 = LSEQ * half
        acc = tuple(rows[par, r0, pl.ds(16 * k, 16)] for k in range(8))

        def red7(j, accs, _r0=r0, _par=par):
          r = _r0 + 1 + j * 7
          for t in range(7):
            accs = tuple(
                accs[k] + rows[_par, r + t, pl.ds(16 * k, 16)]
                for k in range(8))
          return accs

        acc = lax.fori_loop(0, 7, red7, acc)
        row_out = 2 * c_l + half
        for k in range(8):
          stage[row_out, pl.ds(16 * k, 16)] = acc[k] * INV_L

    pltpu.sync_copy(idx_hbm.at[pl.ds(ibase, CHUNKS)], idx_v)

    def start(c, par):
      pltpu.async_copy(
          tbl.at[idx_v.at[c, pl.ds(0, 100)]], rows.at[par], sems[par])

    for p in range(4):
      start(p, p)

    def body(i, carry):
      g = i // 16
      for par in range(4):
        c = 4 * i + par
        pltpu.make_async_copy(
            tbl.at[idx_v.at[c, pl.ds(0, 100)]], rows.at[par],
            sems[par]).wait()
        reduce_bag(par, c - g * 64)

        @pl.when(c + 4 < CHUNKS)
        def _():
          start(c + 4, par)

      @pl.when(i % 16 == 15)
      def _():
        pltpu.sync_copy(stage, out.at[pl.ds(obase + g * 128, 128)])

      return carry

    lax.fori_loop(0, CHUNKS // 4, body, 0)

  return bag(idx_pad, fused)


_BLK = 2048


def _towers_body(up_ref, ip_ref, w1t_ref, b1_ref, w2t_ref, b2_ref, wat_ref,
                 ba_ref, out_ref):
  up = up_ref[...]
  ip = ip_ref[...]
  mlp = jnp.concatenate([up[:, :D], ip[:, :D]], axis=1)
  h1 = jnp.maximum(
      jnp.dot(mlp, w1t_ref[...], preferred_element_type=jnp.float32)
      + b1_ref[...], 0.0)
  h2 = jnp.maximum(
      jnp.dot(h1, w2t_ref[...], preferred_element_type=jnp.float32)
      + b2_ref[...], 0.0)
  gmf = up[:, D:] * ip[:, D:]
  vec = jnp.concatenate([h2, gmf], axis=1)
  out_ref[...] = (
      jnp.dot(vec, wat_ref[...], preferred_element_type=jnp.float32)
      + ba_ref[...])


def _towers(up, ip, W1, b1, W2, b2, Wa, ba):
  grid = B // _BLK
  emb_spec = pl.BlockSpec((_BLK, 2 * D), lambda g: (g, 0))
  full = lambda shape: pl.BlockSpec(shape, lambda g: (0, 0))
  return pl.pallas_call(
      _towers_body,
      grid=(grid,),
      in_specs=[
          emb_spec, emb_spec,
          full((128, 64)), full((1, 64)),
          full((64, 32)), full((1, 32)),
          full((96, 1)), full((1, 1)),
      ],
      out_specs=pl.BlockSpec((_BLK, 1), lambda g: (g, 0)),
      out_shape=jax.ShapeDtypeStruct((B, 1), jnp.float32),
  )(up, ip, W1.T, b1.reshape(1, 64), W2.T, b2.reshape(1, 32),
    Wa.T, ba.reshape(1, 1))


def kernel(usr_comments, descriptions, emb_user_mlp, emb_item_mlp,
           emb_user_gmf, emb_item_gmf, W1, b1, W2, b2, Wa, ba):
  usr_pad = jnp.pad(usr_comments.reshape(B // 2, 2 * LSEQ),
                    ((0, 0), (0, 28)))
  desc_pad = jnp.pad(descriptions.reshape(B // 2, 2 * LSEQ),
                     ((0, 0), (0, 28)))
  fused_u = _fuse_transpose(emb_user_mlp.T, emb_user_gmf.T)
  fused_i = _fuse_transpose(emb_item_mlp.T, emb_item_gmf.T)
  up = _emb_bag(usr_pad, fused_u)
  ip = _emb_bag(desc_pad, fused_i)
  return _towers(up, ip, W1, b1, W2, b2, Wa, ba)
